# Initial kernel scaffold; baseline (speedup 1.0000x reference)
import jax
import jax.numpy as jnp
from jax.experimental import pallas as pl


def kernel(points, feats, kq_key, kq_query, mlp1_w, bn1_g, bn1_b, mlp2_w, bn2_g, bn2_b, mlp3_w, mlp3_b, q_w, q_b, v_w, v_b, out_w, out_b, inter_pos_enc):
    raise NotImplementedError("write your pallas kernel here")


# speedup vs baseline: 30.2947x; 30.2947x over previous
import functools

import jax
import jax.numpy as jnp
from jax import lax
from jax.experimental import pallas as pl
from jax.experimental.pallas import tpu as pltpu
from jax.experimental.pallas import tpu_sc as plsc

N_PTS = 20000
M_PAIRS = 320000
IN_C = 256
OUT_C = 256
H = 8
AC = 32
KV = 27
EPS = 1e-5

NTILES = 16
BATCH = 128
PAIRS_PER_TILE = 20096
NBATCH = PAIRS_PER_TILE // BATCH
M_PAD = PAIRS_PER_TILE * NTILES
ROWS_PER_TILE = 1248
ZCHUNK = 208
EXTRA_ROWS = N_PTS - NTILES * ROWS_PER_TILE
ACC_ROWS = N_PTS + 16


def _k1_body(pts_ref, w1_ref, g1_ref, b1_ref, w2_ref, g2_ref, b2_ref,
             h1_ref, scale2_ref, shift2_ref):
    pts = pts_ref[...]
    w1 = w1_ref[...]
    ipe1 = jnp.dot(pts, w1, preferred_element_type=jnp.float32)
    mu1 = jnp.mean(ipe1, axis=0, keepdims=True)
    var1 = jnp.mean((ipe1 - mu1) ** 2, axis=0, keepdims=True)
    h1 = (ipe1 - mu1) / jnp.sqrt(var1 + EPS) * g1_ref[...] + b1_ref[...]
    h1 = jnp.maximum(h1, 0.0)
    h1_ref[...] = h1
    m1 = jnp.mean(h1, axis=0, keepdims=True)
    hc = h1 - m1
    cov = lax.dot_general(hc, hc, (((0,), (0,)), ((), ())),
                          preferred_element_type=jnp.float32) / N_PTS
    w2 = w2_ref[...]
    mu2 = jnp.dot(m1, w2, preferred_element_type=jnp.float32)
    cw = lax.dot_general(cov, w2, (((1,), (0,)), ((), ())),
                         preferred_element_type=jnp.float32)
    var2 = jnp.sum(w2 * cw, axis=0, keepdims=True)
    scale2 = g2_ref[...] / jnp.sqrt(var2 + EPS)
    shift2 = b2_ref[...] - mu2 * scale2
    scale2_ref[...] = scale2
    shift2_ref[...] = shift2


def _k2_body(h1_ref, feats_ref, scale2_ref, shift2_ref, w2_ref, w3_ref,
             b3_ref, qw_ref, qb_ref, vw_ref, vb_ref, wp_ref, hm_ref,
             hmt_ref, a_ref, vg_ref):
    h2 = jnp.dot(h1_ref[...], w2_ref[...],
                 preferred_element_type=jnp.float32)
    h2 = jnp.maximum(h2 * scale2_ref[...] + shift2_ref[...], 0.0)
    ipe = jnp.dot(h2, w3_ref[...],
                  preferred_element_type=jnp.float32) + b3_ref[...]
    x = feats_ref[...] + ipe
    q = jnp.dot(x, qw_ref[...], preferred_element_type=jnp.float32) + qb_ref[...]
    s = jnp.dot(q * q, hm_ref[...], preferred_element_type=jnp.float32)
    denom = jnp.maximum(jnp.sqrt(s), 1e-12)
    invq = jnp.dot(1.0 / denom, hmt_ref[...],
                   preferred_element_type=jnp.float32)
    nq = q * invq
    a_ref[...] = jnp.dot(nq, wp_ref[...], preferred_element_type=jnp.float32)
    v = jnp.dot(x, vw_ref[...], preferred_element_type=jnp.float32) + vb_ref[...]
    for g in range(4):
        vg_ref[g] = v[:, 64 * g:64 * (g + 1)]


def _k4_body(of_ref, ow_ref, ob_ref, out_ref):
    xf = jnp.concatenate([of_ref[g] for g in range(4)], axis=-1)
    out_ref[...] = jnp.dot(xf, ow_ref[...],
                           preferred_element_type=jnp.float32) + ob_ref[...]


def _sc_pass(g, s_id, a_hbm, vg_hbm, key_hbm, qry_hbm, out_hbm,
             kqb, qb, aib, vib, arows, vrows, contrib, zbuf, acc,
             sem_a, sem_v):
    for j in range(ROWS_PER_TILE // ZCHUNK):
        pltpu.sync_copy(zbuf, acc.at[pl.ds(s_id * ROWS_PER_TILE + j * ZCHUNK,
                                           ZCHUNK), :])

    @pl.when(s_id == NTILES - 1)
    def _zero_tail():
        pltpu.sync_copy(zbuf.at[pl.ds(0, EXTRA_ROWS), :],
                        acc.at[pl.ds(NTILES * ROWS_PER_TILE, EXTRA_ROWS), :])

    plsc.subcore_barrier()

    base0 = s_id * PAIRS_PER_TILE

    def batch_body(b, carry):
        base = base0 + b * BATCH
        pltpu.sync_copy(key_hbm.at[pl.ds(base, BATCH)], kqb)
        pltpu.sync_copy(qry_hbm.at[pl.ds(base, BATCH)], qb)
        for j in range(BATCH // 16):
            kq = kqb[pl.ds(16 * j, 16)]
            qv = qb[pl.ds(16 * j, 16)]
            keyi = (kq.astype(jnp.float32) * (1.0 / KV)
                    + 0.01).astype(jnp.int32)
            kmod = kq - keyi * KV
            ai = jnp.minimum(qv * KV + kmod, N_PTS * KV - 1)
            aib[pl.ds(16 * j, 16)] = ai
            vib[pl.ds(16 * j, 16)] = keyi + g * N_PTS
        cp_a = pltpu.async_copy(a_hbm.at[aib], arows, sem_a)
        cp_v = pltpu.async_copy(vg_hbm.at[vib], vrows, sem_v)
        cp_a.wait()
        cp_v.wait()

        splat0 = jnp.full((16, 1), 2 * g, jnp.int32)
        splat1 = jnp.full((16, 1), 2 * g + 1, jnp.int32)
        gdn = lax.GatherDimensionNumbers(
            offset_dims=(), collapsed_slice_dims=(0,), start_index_map=(0,))

        def _splat(vec, idx):
            return lax.gather(vec, idx, gdn, (1,), unique_indices=False,
                              indices_are_sorted=False,
                              mode=lax.GatherScatterMode.PROMISE_IN_BOUNDS)

        def pair_body(i, c2):
            arow = arows[i, pl.ds(0, 16)]
            s0 = _splat(arow, splat0)
            s1 = _splat(arow, splat1)
            contrib[i, pl.ds(0, 16)] = vrows[i, pl.ds(0, 16)] * s0
            contrib[i, pl.ds(16, 16)] = vrows[i, pl.ds(16, 16)] * s0
            contrib[i, pl.ds(32, 16)] = vrows[i, pl.ds(32, 16)] * s1
            contrib[i, pl.ds(48, 16)] = vrows[i, pl.ds(48, 16)] * s1
            return c2

        lax.fori_loop(0, BATCH, pair_body, 0)
        pltpu.sync_copy(contrib, acc.at[qb], add=True)
        return carry

    lax.fori_loop(0, NBATCH, batch_body, 0)
    plsc.subcore_barrier()
    for j in range(ROWS_PER_TILE // ZCHUNK):
        r = s_id * ROWS_PER_TILE + j * ZCHUNK
        pltpu.sync_copy(acc.at[pl.ds(r, ZCHUNK), :],
                        out_hbm.at[pl.ds(g * N_PTS + r, ZCHUNK), :])

    @pl.when(s_id == NTILES - 1)
    def _read_tail():
        r = NTILES * ROWS_PER_TILE
        pltpu.sync_copy(acc.at[pl.ds(r, EXTRA_ROWS), :],
                        out_hbm.at[pl.ds(g * N_PTS + r, EXTRA_ROWS), :])

    plsc.subcore_barrier()


def _sc_body(a_hbm, vg_hbm, key_hbm, qry_hbm, out_hbm,
             kqb, qb, aib, vib, arows, vrows, contrib, zbuf, acc,
             sem_a, sem_v):
    s_id = lax.axis_index("s")
    c_id = lax.axis_index("c")

    z16 = jnp.zeros((16,), jnp.float32)

    def zrow(i, carry):
        for j in range(4):
            zbuf[i, pl.ds(16 * j, 16)] = z16
        return carry

    lax.fori_loop(0, ZCHUNK, zrow, 0)

    for p in range(2):
        for cv in range(2):
            gid = 2 * p + cv

            @pl.when(c_id == cv)
            def _run(gid=gid):
                _sc_pass(gid, s_id, a_hbm, vg_hbm, key_hbm, qry_hbm, out_hbm,
                         kqb, qb, aib, vib, arows, vrows, contrib, zbuf, acc,
                         sem_a, sem_v)


def kernel(points, feats, kq_key, kq_query, mlp1_w, bn1_g, bn1_b, mlp2_w,
           bn2_g, bn2_b, mlp3_w, mlp3_b, q_w, q_b, v_w, v_b, out_w, out_b,
           inter_pos_enc):
    f32 = jnp.float32
    g1 = bn1_g.reshape(1, 3)
    b1 = bn1_b.reshape(1, 3)
    g2 = bn2_g.reshape(1, IN_C)
    b2 = bn2_b.reshape(1, IN_C)
    b3 = mlp3_b.reshape(1, IN_C)
    qb2 = q_b.reshape(1, OUT_C)
    vb2 = v_b.reshape(1, OUT_C)
    ob2 = out_b.reshape(1, OUT_C)
    pe = inter_pos_enc
    pen = pe / jnp.maximum(
        jnp.sqrt(jnp.sum(pe * pe, axis=-1, keepdims=True)), 1e-12)
    kk, hh, cc = jnp.meshgrid(jnp.arange(KV), jnp.arange(H), jnp.arange(AC),
                              indexing="ij")
    w2p = jnp.zeros((OUT_C, KV * 16), f32).at[
        (hh * AC + cc).ravel(), (kk * 16 + hh).ravel()].set(pen.ravel())
    hm = (jnp.arange(OUT_C)[:, None] // AC
          == jnp.arange(H)[None, :]).astype(f32)
    hmt = hm.T
    pad = M_PAD - M_PAIRS
    kq_key_p = jnp.concatenate([kq_key, jnp.zeros((pad,), jnp.int32)])
    kq_query_p = jnp.concatenate(
        [kq_query, jnp.full((pad,), N_PTS, jnp.int32)])

    h1_n, scale2, shift2 = pl.pallas_call(
        _k1_body,
        out_shape=(
            jax.ShapeDtypeStruct((N_PTS, 3), f32),
            jax.ShapeDtypeStruct((1, IN_C), f32),
            jax.ShapeDtypeStruct((1, IN_C), f32),
        ),
    )(points, mlp1_w, g1, b1, mlp2_w, g2, b2)

    NB = 2000
    grid = (N_PTS // NB,)
    a2, vg = pl.pallas_call(
        _k2_body,
        grid=grid,
        in_specs=[
            pl.BlockSpec((NB, 3), lambda i: (i, 0)),
            pl.BlockSpec((NB, IN_C), lambda i: (i, 0)),
            pl.BlockSpec((1, IN_C), lambda i: (0, 0)),
            pl.BlockSpec((1, IN_C), lambda i: (0, 0)),
            pl.BlockSpec((3, IN_C), lambda i: (0, 0)),
            pl.BlockSpec((IN_C, IN_C), lambda i: (0, 0)),
            pl.BlockSpec((1, IN_C), lambda i: (0, 0)),
            pl.BlockSpec((IN_C, OUT_C), lambda i: (0, 0)),
            pl.BlockSpec((1, OUT_C), lambda i: (0, 0)),
            pl.BlockSpec((IN_C, OUT_C), lambda i: (0, 0)),
            pl.BlockSpec((1, OUT_C), lambda i: (0, 0)),
            pl.BlockSpec((OUT_C, KV * 16), lambda i: (0, 0)),
            pl.BlockSpec((OUT_C, H), lambda i: (0, 0)),
            pl.BlockSpec((H, OUT_C), lambda i: (0, 0)),
        ],
        out_specs=(
            pl.BlockSpec((NB, KV * 16), lambda i: (i, 0)),
            pl.BlockSpec((4, NB, 64), lambda i: (0, i, 0)),
        ),
        out_shape=(
            jax.ShapeDtypeStruct((N_PTS, KV * 16), f32),
            jax.ShapeDtypeStruct((4, N_PTS, 64), f32),
        ),
    )(h1_n, feats, scale2, shift2, mlp2_w, mlp3_w, b3, q_w, qb2, v_w, vb2,
      w2p, hm, hmt)

    a_pad = a2.reshape(N_PTS * KV, 16)
    vg_flat = vg.reshape(4 * N_PTS, 64)

    mesh = plsc.VectorSubcoreMesh(core_axis_name="c", subcore_axis_name="s")
    sc_fn = pl.kernel(
        _sc_body,
        out_type=jax.ShapeDtypeStruct((4 * N_PTS, 64), f32),
        mesh=mesh,
        scratch_types=[
            pltpu.VMEM((BATCH,), jnp.int32),
            pltpu.VMEM((BATCH,), jnp.int32),
            pltpu.VMEM((BATCH,), jnp.int32),
            pltpu.VMEM((BATCH,), jnp.int32),
            pltpu.VMEM((BATCH, 16), f32),
            pltpu.VMEM((BATCH, 64), f32),
            pltpu.VMEM((BATCH, 64), f32),
            pltpu.VMEM((ZCHUNK, 64), f32),
            pltpu.VMEM_SHARED((ACC_ROWS, 64), f32),
            pltpu.SemaphoreType.DMA,
            pltpu.SemaphoreType.DMA,
        ],
        compiler_params=pltpu.CompilerParams(use_tc_tiling_on_sc=False),
    )
    out_sc = sc_fn(a_pad, vg_flat, kq_key_p, kq_query_p)

    of = out_sc.reshape(4, N_PTS, 64)
    out = pl.pallas_call(
        _k4_body,
        grid=grid,
        in_specs=[
            pl.BlockSpec((4, NB, 64), lambda i: (0, i, 0)),
            pl.BlockSpec((OUT_C, OUT_C), lambda i: (0, 0)),
            pl.BlockSpec((1, OUT_C), lambda i: (0, 0)),
        ],
        out_specs=pl.BlockSpec((NB, OUT_C), lambda i: (i, 0)),
        out_shape=jax.ShapeDtypeStruct((N_PTS, OUT_C), f32),
    )(of, out_w, ob2)
    return out
